# baseline (speedup 1.0000x reference)
import functools

import jax
import jax.numpy as jnp
from jax import lax
from jax.experimental import pallas as pl
from jax.experimental.pallas import tpu as pltpu
from jax.experimental.pallas import tpu_sc as plsc

N_GRAMS = 20
N_VOCAB = 100000
EMB = 32
HID = 64
BATCH = 1024
N_IDX = BATCH * N_GRAMS

BM = 16
NM = BATCH // BM
NBUF = 4

_IDX_CHUNK = 128


def _sc_gather(table, idx3):
    info = plsc.get_sparse_core_info()
    nw = info.num_cores * info.num_subcores
    b_per_w = N_IDX // nw
    n_ch = b_per_w // _IDX_CHUNK
    mesh = plsc.VectorSubcoreMesh(core_axis_name="c", subcore_axis_name="s")

    @functools.partial(
        pl.kernel,
        mesh=mesh,
        out_type=jax.ShapeDtypeStruct((N_IDX, EMB), jnp.float32),
        scratch_types=[
            pltpu.VMEM((n_ch, _IDX_CHUNK), jnp.int32),
            pltpu.VMEM((b_per_w, EMB), jnp.float32),
            pltpu.SemaphoreType.DMA,
        ],
        compiler_params=pltpu.CompilerParams(use_tc_tiling_on_sc=False),
    )
    def k(table_hbm, idx_hbm, out_hbm, idx_v, rows_v, sem):
        wid = lax.axis_index("s") * info.num_cores + lax.axis_index("c")
        base = wid * b_per_w
        pltpu.sync_copy(idx_hbm.at[wid], idx_v)
        copies = [
            pltpu.make_async_copy(
                table_hbm.at[idx_v.at[j]],
                rows_v.at[pl.ds(j * _IDX_CHUNK, _IDX_CHUNK)],
                sem,
            )
            for j in range(n_ch)
        ]
        for c in copies:
            c.start()
        for c in copies:
            c.wait()
        pltpu.sync_copy(rows_v, out_hbm.at[pl.ds(base, b_per_w)])

    return k(table, idx3)


def _body(cat_ref, w0_ref, b0_ref, w1t_ref, b1_ref, out_ref, out1_ref, buf, sems):
    i = pl.program_id(0)
    slot = lax.rem(i, NBUF)

    @pl.when(i == 0)
    def _():
        h = lax.dot_general(
            cat_ref[...].astype(jnp.bfloat16),
            w0_ref[...].astype(jnp.bfloat16),
            (((1,), (1,)), ((), ())),
            preferred_element_type=jnp.float32,
        )
        out1_ref[...] = jax.nn.relu(h + b0_ref[...])

    def _copy(k, ds_i):
        return pltpu.make_async_copy(
            buf.at[k], out_ref.at[pl.ds(ds_i * BM, BM)], sems.at[k]
        )

    @pl.when(i >= NBUF)
    def _():
        _copy(slot, i - NBUF).wait()

    logits = (
        jnp.dot(
            out1_ref[pl.ds(i * BM, BM), :].astype(jnp.bfloat16),
            w1t_ref[...],
            preferred_element_type=jnp.float32,
        )
        + b1_ref[...]
    )
    m = jnp.max(logits, axis=1, keepdims=True)
    p = jnp.exp(logits - m)
    s_inv = 1.0 / jnp.sum(p, axis=1, keepdims=True)
    buf[slot] = p * s_inv
    _copy(slot, i).start()

    @pl.when(i == NM - 1)
    def _():
        for k in range(NBUF):
            _copy(k, 0).wait()


def kernel(x, table, W0, b0, W1, b1):
    idx3 = x.reshape(-1).reshape(32, N_IDX // 32 // _IDX_CHUNK, _IDX_CHUNK)
    rows = _sc_gather(table, idx3)
    cat = rows.reshape(BATCH, N_GRAMS * EMB)

    b0r = b0.reshape(1, HID)
    b1r = b1.reshape(1, N_VOCAB)
    w1t = W1.T.astype(jnp.bfloat16)

    whole = lambda shape: pl.BlockSpec(shape, lambda j: (0,) * len(shape))

    out = pl.pallas_call(
        _body,
        grid=(NM,),
        in_specs=[
            whole((BATCH, N_GRAMS * EMB)),
            whole((HID, N_GRAMS * EMB)),
            whole((1, HID)),
            whole((HID, N_VOCAB)),
            whole((1, N_VOCAB)),
        ],
        out_specs=pl.BlockSpec(memory_space=pl.ANY),
        out_shape=jax.ShapeDtypeStruct((BATCH, N_VOCAB), jnp.float32),
        scratch_shapes=[
            pltpu.VMEM((BATCH, HID), jnp.float32),
            pltpu.VMEM((NBUF, BM, N_VOCAB), jnp.float32),
            pltpu.SemaphoreType.DMA((NBUF,)),
        ],
        compiler_params=pltpu.CompilerParams(
            dimension_semantics=("arbitrary",),
        ),
    )(cat, W0, b0r, w1t, b1r)
    return out


# speedup vs baseline: 1.0838x; 1.0838x over previous
import functools

import jax
import jax.numpy as jnp
from jax import lax
from jax.experimental import pallas as pl
from jax.experimental.pallas import tpu as pltpu
from jax.experimental.pallas import tpu_sc as plsc

N_GRAMS = 20
N_VOCAB = 100000
EMB = 32
HID = 64
BATCH = 1024
N_IDX = BATCH * N_GRAMS

BM = 32
NM = BATCH // BM
NBUF = 2

_IDX_CHUNK = 128


def _sc_gather(table, idx3):
    info = plsc.get_sparse_core_info()
    nw = info.num_cores * info.num_subcores
    b_per_w = N_IDX // nw
    n_ch = b_per_w // _IDX_CHUNK
    mesh = plsc.VectorSubcoreMesh(core_axis_name="c", subcore_axis_name="s")

    @functools.partial(
        pl.kernel,
        mesh=mesh,
        out_type=jax.ShapeDtypeStruct((N_IDX, EMB), jnp.float32),
        scratch_types=[
            pltpu.VMEM((n_ch, _IDX_CHUNK), jnp.int32),
            pltpu.VMEM((b_per_w, EMB), jnp.float32),
            pltpu.SemaphoreType.DMA,
        ],
        compiler_params=pltpu.CompilerParams(use_tc_tiling_on_sc=False),
    )
    def k(table_hbm, idx_hbm, out_hbm, idx_v, rows_v, sem):
        wid = lax.axis_index("s") * info.num_cores + lax.axis_index("c")
        base = wid * b_per_w
        pltpu.sync_copy(idx_hbm.at[wid], idx_v)
        copies = [
            pltpu.make_async_copy(
                table_hbm.at[idx_v.at[j]],
                rows_v.at[pl.ds(j * _IDX_CHUNK, _IDX_CHUNK)],
                sem,
            )
            for j in range(n_ch)
        ]
        for c in copies:
            c.start()
        for c in copies:
            c.wait()
        pltpu.sync_copy(rows_v, out_hbm.at[pl.ds(base, b_per_w)])

    return k(table, idx3)


def _body(cat_ref, w0_ref, b0_ref, w1t_ref, b1_ref, out_ref, out1_ref, buf, sems):
    i = pl.program_id(0)
    slot = lax.rem(i, NBUF)

    @pl.when(i == 0)
    def _():
        h = lax.dot_general(
            cat_ref[...].astype(jnp.bfloat16),
            w0_ref[...].astype(jnp.bfloat16),
            (((1,), (1,)), ((), ())),
            preferred_element_type=jnp.float32,
        )
        out1_ref[...] = jax.nn.relu(h + b0_ref[...])

    def _copy(k, ds_i):
        return pltpu.make_async_copy(
            buf.at[k], out_ref.at[pl.ds(ds_i * BM, BM)], sems.at[k]
        )

    @pl.when(i >= NBUF)
    def _():
        _copy(slot, i - NBUF).wait()

    logits = (
        jnp.dot(
            out1_ref[pl.ds(i * BM, BM), :].astype(jnp.bfloat16),
            w1t_ref[...],
            preferred_element_type=jnp.float32,
        )
        + b1_ref[...]
    )
    m = jnp.max(logits, axis=1, keepdims=True)
    p = jnp.exp(logits - m)
    s_inv = 1.0 / jnp.sum(p, axis=1, keepdims=True)
    buf[slot] = p * s_inv
    _copy(slot, i).start()

    @pl.when(i == NM - 1)
    def _():
        for k in range(NBUF):
            _copy(k, 0).wait()


def kernel(x, table, W0, b0, W1, b1):
    idx3 = x.reshape(-1).reshape(32, N_IDX // 32 // _IDX_CHUNK, _IDX_CHUNK)
    rows = _sc_gather(table, idx3)
    cat = rows.reshape(BATCH, N_GRAMS * EMB)

    b0r = b0.reshape(1, HID)
    b1r = b1.reshape(1, N_VOCAB)
    w1t = W1.T.astype(jnp.bfloat16)

    whole = lambda shape: pl.BlockSpec(shape, lambda j: (0,) * len(shape))

    out = pl.pallas_call(
        _body,
        grid=(NM,),
        in_specs=[
            whole((BATCH, N_GRAMS * EMB)),
            whole((HID, N_GRAMS * EMB)),
            whole((1, HID)),
            whole((HID, N_VOCAB)),
            whole((1, N_VOCAB)),
        ],
        out_specs=pl.BlockSpec(memory_space=pl.ANY),
        out_shape=jax.ShapeDtypeStruct((BATCH, N_VOCAB), jnp.float32),
        scratch_shapes=[
            pltpu.VMEM((BATCH, HID), jnp.float32),
            pltpu.VMEM((NBUF, BM, N_VOCAB), jnp.float32),
            pltpu.SemaphoreType.DMA((NBUF,)),
        ],
        compiler_params=pltpu.CompilerParams(
            dimension_semantics=("arbitrary",),
        ),
    )(cat, W0, b0r, w1t, b1r)
    return out
